# Initial kernel scaffold; baseline (speedup 1.0000x reference)
import jax
import jax.numpy as jnp
from jax.experimental import pallas as pl


def kernel(X, pos, target, conv_W, conv_b, norm_g, norm_b, dist_W1, dist_b1, dist_W2, dist_b2):
    raise NotImplementedError("write your pallas kernel here")


# speedup vs baseline: 2.5800x; 2.5800x over previous
import functools

import jax
import jax.numpy as jnp
from jax import lax
from jax.experimental import pallas as pl
from jax.experimental.pallas import tpu as pltpu
from jax.experimental.pallas import tpu_sc as plsc

F32 = jnp.float32
I32 = jnp.int32

N = 100000
NTOT = 100352
T = 16384
KK = 32
BM = 1024
NW = 32
CH = NTOT // NW
WIN = 392
NWIN = CH // WIN
TCH = T // NW
BT = 2048
FP = 136


def _proj_body(x_ref, w_ref, cb_ref, a_ref, bb_ref, c_ref):
    m = pl.program_id(0)
    y = jnp.dot(x_ref[...], w_ref[...], preferred_element_type=F32)
    rows = m * BM + lax.broadcasted_iota(I32, (BM, 1), 0)
    msk = rows < N
    a_ref[...] = jnp.where(msk, y[:, 0:KK], 0.0)
    bb_ref[...] = jnp.where(msk, 4.0 * (y[:, KK:2 * KK] + cb_ref[...]), 0.0)
    c_ref[...] = jnp.where(msk, y[:, 2 * KK:3 * KK], 0.0)


def _proj_ln_body(x_ref, w_ref, cb_ref, g_ref, nb_ref, a_ref, bb_ref, c_ref):
    m = pl.program_id(0)
    x = jax.nn.relu(x_ref[...])
    mu = jnp.mean(x, axis=-1, keepdims=True)
    va = jnp.var(x, axis=-1, keepdims=True)
    x = (x - mu) * lax.rsqrt(va + 1e-3) * g_ref[...] + nb_ref[...]
    y = jnp.dot(x, w_ref[...], preferred_element_type=F32)
    rows = m * BM + lax.broadcasted_iota(I32, (BM, 1), 0)
    msk = rows < N
    a_ref[...] = jnp.where(msk, y[:, 0:KK], 0.0)
    bb_ref[...] = jnp.where(msk, 4.0 * (y[:, KK:2 * KK] + cb_ref[...]), 0.0)
    c_ref[...] = jnp.where(msk, y[:, 2 * KK:3 * KK], 0.0)


def _proj(x, wcat, cb, g=None, nb=None):
    cin = x.shape[1]
    grid = NTOT // BM
    outs = [jax.ShapeDtypeStruct((NTOT, KK), F32)] * 3
    full = lambda r, c: pl.BlockSpec((r, c), lambda m: (0, 0))
    blk = lambda c: pl.BlockSpec((BM, c), lambda m: (m, 0))
    if g is None:
        return pl.pallas_call(
            _proj_body, grid=(grid,),
            in_specs=[blk(cin), full(cin, 3 * KK), full(1, KK)],
            out_specs=[blk(KK)] * 3,
            out_shape=outs,
        )(x, wcat, cb)
    return pl.pallas_call(
        _proj_ln_body, grid=(grid,),
        in_specs=[blk(cin), full(cin, 3 * KK), full(1, KK),
                  full(1, KK), full(1, KK)],
        out_specs=[blk(KK)] * 3,
        out_shape=outs,
    )(x, wcat, cb, g, nb)


@functools.lru_cache(maxsize=None)
def _sc_mesh():
    return plsc.VectorSubcoreMesh(core_axis_name="c", subcore_axis_name="s")


@functools.lru_cache(maxsize=None)
def _gather_acc_kernel():
    return functools.partial(
        pl.kernel,
        out_type=jax.ShapeDtypeStruct((NTOT, KK), F32),
        mesh=_sc_mesh(),
        compiler_params=pltpu.CompilerParams(use_tc_tiling_on_sc=False),
        scratch_types=(
            [pltpu.VMEM((WIN,), I32) for _ in range(8)]
            + [pltpu.VMEM((WIN, KK), F32) for _ in range(8)]
            + [pltpu.VMEM((WIN, KK), F32), pltpu.SemaphoreType.DMA]
        ),
    )(_gather_acc_body)


def _gather_acc_body(a_hbm, c_hbm, bb_hbm,
                x0_hbm, x1_hbm, x2_hbm, x3_hbm,
                x4_hbm, x5_hbm, x6_hbm, x7_hbm, out_hbm,
                i0, i1, i2, i3, i4, i5, i6, i7,
                g0, g1, g2, g3, g4, g5, g6, g7, accv, sem):
    wid = lax.axis_index("s") * 2 + lax.axis_index("c")
    idx_hbms = (x0_hbm, x1_hbm, x2_hbm, x3_hbm, x4_hbm, x5_hbm, x6_hbm, x7_hbm)
    ivs = (i0, i1, i2, i3, i4, i5, i6, i7)
    gvs = (g0, g1, g2, g3, g4, g5, g6, g7)
    for k in range(NWIN):
        base = wid * CH + k * WIN
        pltpu.sync_copy(bb_hbm.at[pl.ds(base, WIN)], accv)
        for t in range(8):
            pltpu.sync_copy(idx_hbms[t].at[pl.ds(base, WIN)], ivs[t])
        cps = []
        for t in range(8):
            tab = a_hbm if t % 2 == 0 else c_hbm
            cps.append(pltpu.async_copy(tab.at[ivs[t]], gvs[t], sem))
        for cp in cps:
            cp.wait()

        def row_body(r, _):
            for h in (0, 16):
                v = accv[r, pl.ds(h, 16)]
                for gv in gvs:
                    v = v + gv[r, pl.ds(h, 16)]
                accv[r, pl.ds(h, 16)] = v
            return 0

        lax.fori_loop(0, WIN, row_body, 0)
        pltpu.sync_copy(accv, out_hbm.at[pl.ds(base, WIN)])


@functools.lru_cache(maxsize=None)
def _target_gather_kernel():
    return functools.partial(
        pl.kernel,
        out_type=(jax.ShapeDtypeStruct((T, FP), F32),
                  jax.ShapeDtypeStruct((T, KK), F32),
                  jax.ShapeDtypeStruct((T, KK), F32),
                  jax.ShapeDtypeStruct((T, KK), F32)),
        mesh=_sc_mesh(),
        compiler_params=pltpu.CompilerParams(use_tc_tiling_on_sc=False),
        scratch_types=(pltpu.VMEM((TCH,), I32),
                       pltpu.VMEM((TCH, FP), F32),
                       pltpu.VMEM((TCH, KK), F32),
                       pltpu.SemaphoreType.DMA),
    )(_target_gather_body)


def _target_gather_body(x_hbm, a1_hbm, a2_hbm, a3_hbm, tgt_hbm,
                   x_out, t1_out, t2_out, t3_out, idx_v, x_v, t_v, sem):
    wid = lax.axis_index("s") * 2 + lax.axis_index("c")
    base = wid * TCH
    pltpu.sync_copy(tgt_hbm.at[pl.ds(base, TCH)], idx_v)
    pltpu.async_copy(x_hbm.at[idx_v], x_v, sem).wait()
    pltpu.sync_copy(x_v, x_out.at[pl.ds(base, TCH)])
    for tab, out in ((a1_hbm, t1_out), (a2_hbm, t2_out), (a3_hbm, t3_out)):
        pltpu.async_copy(tab.at[idx_v], t_v, sem).wait()
        pltpu.sync_copy(t_v, out.at[pl.ds(base, TCH)])


def _softplus(x):
    return jnp.maximum(x, 0.0) + jnp.log(1.0 + jnp.exp(-jnp.abs(x)))


def _dist_body(x0_ref, t1_ref, t2_ref, t3_ref,
               w10_ref, b10_ref, w20_ref, b20_ref,
               g1_ref, nb1_ref, w11_ref, b11_ref, w21_ref, b21_ref,
               g2_ref, nb2_ref, w12_ref, b12_ref, w22_ref, b22_ref,
               g3_ref, nb3_ref, w13_ref, b13_ref, w23_ref, b23_ref,
               out_ref):
    h = _softplus(jnp.dot(x0_ref[...], w10_ref[...],
                          preferred_element_type=F32) + b10_ref[...])
    y = jnp.dot(h, w20_ref[...], preferred_element_type=F32) + b20_ref[...]
    for t_ref, g_ref, nb_ref, w1_ref, b1_ref, w2_ref, b2_ref in (
            (t1_ref, g1_ref, nb1_ref, w11_ref, b11_ref, w21_ref, b21_ref),
            (t2_ref, g2_ref, nb2_ref, w12_ref, b12_ref, w22_ref, b22_ref),
            (t3_ref, g3_ref, nb3_ref, w13_ref, b13_ref, w23_ref, b23_ref)):
        x = jax.nn.relu(t_ref[...])
        mu = jnp.mean(x, axis=-1, keepdims=True)
        va = jnp.var(x, axis=-1, keepdims=True)
        x = (x - mu) * lax.rsqrt(va + 1e-3) * g_ref[...] + nb_ref[...]
        h = _softplus(jnp.dot(x, w1_ref[...],
                              preferred_element_type=F32) + b1_ref[...])
        y = y + jnp.dot(h, w2_ref[...],
                        preferred_element_type=F32) + b2_ref[...]
    m = jnp.max(y, axis=-1, keepdims=True)
    e = jnp.exp(y - m)
    out_ref[...] = e / jnp.sum(e, axis=-1, keepdims=True)


def kernel(X, pos, target, conv_W, conv_b, norm_g, norm_b,
           dist_W1, dist_b1, dist_W2, dist_b2):
    n, P, C = X.shape
    DIMS = pos.shape[-1]
    Xc0 = jnp.pad(X.reshape(n, P * C), ((0, 0), (0, FP - P * C)))

    pos32 = pos.astype(I32)
    offset = jnp.array([1, 3, 3], I32)
    shifts = (jnp.arange(DIMS) * P).astype(I32)
    perms = []
    for i in range(DIMS):
        I = jnp.roll(pos32, shift=i, axis=-1) // offset
        key = jnp.sum(I << shifts, axis=-1)
        perms.append(jnp.argsort(key).astype(I32))

    S = n
    ar = jnp.arange(NTOT, dtype=I32)
    idx_list = [jnp.where((ar >= 1) & (ar < n), ar - 1, S),
                jnp.where(ar < n - 1, ar + 1, S)]
    for p in perms:
        lv = jnp.concatenate([jnp.full((1,), S, I32), p[:-1]])
        rv = jnp.concatenate([p[1:], jnp.full((1,), S, I32)])
        idx_list.append(jnp.full((NTOT,), S, I32).at[p].set(lv))
        idx_list.append(jnp.full((NTOT,), S, I32).at[p].set(rv))

    accs = []
    x = Xc0
    for i in range(3):
        wcat = jnp.concatenate([conv_W[i][0], conv_W[i][1], conv_W[i][2]],
                               axis=-1)
        if i == 0:
            wcat = jnp.pad(wcat, ((0, FP - P * C), (0, 0)))
        cb = conv_b[i].reshape(1, KK)
        if i == 0:
            a, bb, c = _proj(x, wcat, cb)
        else:
            a, bb, c = _proj(x, wcat, cb, norm_g[i - 1].reshape(1, KK),
                             norm_b[i - 1].reshape(1, KK))
        acc = _gather_acc_kernel()(a, c, bb, *idx_list)
        accs.append(acc)
        x = acc

    xt0, t1, t2, t3 = _target_gather_kernel()(Xc0, accs[0], accs[1], accs[2],
                                              target)

    grid = T // BT
    full = lambda r, c: pl.BlockSpec((r, c), lambda m: (0, 0))
    blk = lambda c: pl.BlockSpec((BT, c), lambda m: (m, 0))
    args = [xt0, t1, t2, t3,
            jnp.pad(dist_W1[0], ((0, FP - P * C), (0, 0))),
            dist_b1[0].reshape(1, -1),
            dist_W2[0], dist_b2[0].reshape(1, -1)]
    specs = [blk(FP), blk(KK), blk(KK), blk(KK),
             full(FP, 64), full(1, 64), full(64, 2), full(1, 2)]
    for i in range(3):
        args += [norm_g[i].reshape(1, KK), norm_b[i].reshape(1, KK),
                 dist_W1[i + 1], dist_b1[i + 1].reshape(1, -1),
                 dist_W2[i + 1], dist_b2[i + 1].reshape(1, -1)]
        specs += [full(1, KK), full(1, KK),
                  full(KK, 64), full(1, 64), full(64, 2), full(1, 2)]
    out = pl.pallas_call(
        _dist_body, grid=(grid,),
        in_specs=specs,
        out_specs=blk(2),
        out_shape=jax.ShapeDtypeStruct((T, 2), F32),
    )(*args)
    return out


# speedup vs baseline: 5.0069x; 1.9406x over previous
import functools

import jax
import jax.numpy as jnp
from jax import lax
from jax.experimental import pallas as pl
from jax.experimental.pallas import tpu as pltpu
from jax.experimental.pallas import tpu_sc as plsc

F32 = jnp.float32
I32 = jnp.int32

N = 100000
NTOT = 100352
T = 16384
KK = 32
BM = 1024
NW = 32
CH = NTOT // NW
WIN = 392
NWIN = CH // WIN
TCH = T // NW
BT = 2048
FP = 136


def _proj_body(x_ref, w_ref, cb_ref, db_ref, a_ref, bb_ref, c_ref, h_ref):
    m = pl.program_id(0)
    y = jnp.dot(x_ref[...], w_ref[...], preferred_element_type=F32)
    rows = m * BM + lax.broadcasted_iota(I32, (BM, 1), 0)
    msk = rows < N
    a_ref[...] = jnp.where(msk, y[:, 0:KK], 0.0)
    bb_ref[...] = jnp.where(msk, 4.0 * (y[:, KK:2 * KK] + cb_ref[...]), 0.0)
    c_ref[...] = jnp.where(msk, y[:, 2 * KK:3 * KK], 0.0)
    h_ref[...] = y[:, 3 * KK:3 * KK + 64] + db_ref[...]


def _proj_ln_body(x_ref, w_ref, cb_ref, g_ref, nb_ref, a_ref, bb_ref, c_ref):
    m = pl.program_id(0)
    x = jax.nn.relu(x_ref[...])
    mu = jnp.mean(x, axis=-1, keepdims=True)
    va = jnp.var(x, axis=-1, keepdims=True)
    x = (x - mu) * lax.rsqrt(va + 1e-3) * g_ref[...] + nb_ref[...]
    y = jnp.dot(x, w_ref[...], preferred_element_type=F32)
    rows = m * BM + lax.broadcasted_iota(I32, (BM, 1), 0)
    msk = rows < N
    a_ref[...] = jnp.where(msk, y[:, 0:KK], 0.0)
    bb_ref[...] = jnp.where(msk, 4.0 * (y[:, KK:2 * KK] + cb_ref[...]), 0.0)
    c_ref[...] = jnp.where(msk, y[:, 2 * KK:3 * KK], 0.0)


def _proj(x, wcat, cb, g=None, nb=None, db=None):
    cin = x.shape[1]
    grid = NTOT // BM
    outs = [jax.ShapeDtypeStruct((NTOT, KK), F32)] * 3
    full = lambda r, c: pl.BlockSpec((r, c), lambda m: (0, 0))
    blk = lambda c: pl.BlockSpec((BM, c), lambda m: (m, 0))
    if g is None:
        return pl.pallas_call(
            _proj_body, grid=(grid,),
            in_specs=[blk(cin), full(cin, 3 * KK + 64), full(1, KK),
                      full(1, 64)],
            out_specs=[blk(KK)] * 3 + [blk(64)],
            out_shape=outs + [jax.ShapeDtypeStruct((NTOT, 64), F32)],
        )(x, wcat, cb, db)
    return pl.pallas_call(
        _proj_ln_body, grid=(grid,),
        in_specs=[blk(cin), full(cin, 3 * KK), full(1, KK),
                  full(1, KK), full(1, KK)],
        out_specs=[blk(KK)] * 3,
        out_shape=outs,
    )(x, wcat, cb, g, nb)


@functools.lru_cache(maxsize=None)
def _sc_mesh():
    return plsc.VectorSubcoreMesh(core_axis_name="c", subcore_axis_name="s")


PEXT = NTOT + 16
NPV = CH // 16


def _nbr_body(p1, p2, p3, l1, r1, l2, r2, l3, r3, ext, pv, lv, rv, sbuf, sem):
    wid = lax.axis_index("s") * 2 + lax.axis_index("c")
    base = wid * CH
    svec = jnp.full((16,), N, I32)

    @pl.when(wid == 0)
    def _():
        for t in range((NTOT - N) // 16):
            sbuf[pl.ds(t * 16, 16)] = svec
        for out in (l1, r1, l2, r2, l3, r3):
            pltpu.sync_copy(sbuf, out.at[pl.ds(N, NTOT - N)])

    iota = lax.iota(I32, 16)
    for p_hbm, lo, ro in ((p1, l1, r1), (p2, l2, r2), (p3, l3, r3)):
        pltpu.sync_copy(p_hbm.at[pl.ds(base, CH + 16)], ext)
        pltpu.sync_copy(p_hbm.at[pl.ds(base + 8, CH)], pv)

        def body(t, _):
            k0 = t * 16
            lv[pl.ds(k0, 16)] = plsc.load_gather(ext, [iota + (k0 + 7)])
            rv[pl.ds(k0, 16)] = plsc.load_gather(ext, [iota + (k0 + 9)])
            return 0

        lax.fori_loop(0, NPV, body, 0)
        cl = pltpu.async_copy(lv, lo.at[pv], sem)
        cr = pltpu.async_copy(rv, ro.at[pv], sem)
        cl.wait()
        cr.wait()


@functools.lru_cache(maxsize=None)
def _nbr_kernel():
    return functools.partial(
        pl.kernel,
        out_type=tuple(jax.ShapeDtypeStruct((NTOT,), I32) for _ in range(6)),
        mesh=_sc_mesh(),
        compiler_params=pltpu.CompilerParams(use_tc_tiling_on_sc=False,
                                             needs_layout_passes=False),
        scratch_types=(pltpu.VMEM((CH + 16,), I32),
                       pltpu.VMEM((CH,), I32),
                       pltpu.VMEM((CH,), I32),
                       pltpu.VMEM((CH,), I32),
                       pltpu.VMEM((NTOT - N,), I32),
                       pltpu.SemaphoreType.DMA),
    )(_nbr_body)


@functools.lru_cache(maxsize=None)
def _gather_acc_kernel():
    return functools.partial(
        pl.kernel,
        out_type=jax.ShapeDtypeStruct((NTOT, KK), F32),
        mesh=_sc_mesh(),
        compiler_params=pltpu.CompilerParams(use_tc_tiling_on_sc=False),
        scratch_types=(
            [pltpu.VMEM((WIN,), I32) for _ in range(8)]
            + [pltpu.VMEM((WIN, KK), F32) for _ in range(8)]
            + [pltpu.VMEM((WIN, KK), F32), pltpu.SemaphoreType.DMA]
        ),
    )(_gather_acc_body)


def _gather_acc_body(a_hbm, c_hbm, bb_hbm,
                x0_hbm, x1_hbm, x2_hbm, x3_hbm,
                x4_hbm, x5_hbm, x6_hbm, x7_hbm, out_hbm,
                i0, i1, i2, i3, i4, i5, i6, i7,
                g0, g1, g2, g3, g4, g5, g6, g7, accv, sem):
    wid = lax.axis_index("s") * 2 + lax.axis_index("c")
    idx_hbms = (x0_hbm, x1_hbm, x2_hbm, x3_hbm, x4_hbm, x5_hbm, x6_hbm, x7_hbm)
    ivs = (i0, i1, i2, i3, i4, i5, i6, i7)
    gvs = (g0, g1, g2, g3, g4, g5, g6, g7)
    for k in range(NWIN):
        base = wid * CH + k * WIN
        pltpu.sync_copy(bb_hbm.at[pl.ds(base, WIN)], accv)
        for t in range(8):
            pltpu.sync_copy(idx_hbms[t].at[pl.ds(base, WIN)], ivs[t])
        cps = []
        for t in range(8):
            tab = a_hbm if t % 2 == 0 else c_hbm
            cps.append(pltpu.async_copy(tab.at[ivs[t]], gvs[t], sem))
        for cp in cps:
            cp.wait()

        def row_body(r, _):
            for h in (0, 16):
                v = accv[r, pl.ds(h, 16)]
                for gv in gvs:
                    v = v + gv[r, pl.ds(h, 16)]
                accv[r, pl.ds(h, 16)] = v
            return 0

        lax.fori_loop(0, WIN, row_body, 0)
        pltpu.sync_copy(accv, out_hbm.at[pl.ds(base, WIN)])


@functools.lru_cache(maxsize=None)
def _target_gather_kernel():
    return functools.partial(
        pl.kernel,
        out_type=(jax.ShapeDtypeStruct((T, 64), F32),
                  jax.ShapeDtypeStruct((T, KK), F32),
                  jax.ShapeDtypeStruct((T, KK), F32),
                  jax.ShapeDtypeStruct((T, KK), F32)),
        mesh=_sc_mesh(),
        compiler_params=pltpu.CompilerParams(use_tc_tiling_on_sc=False),
        scratch_types=(pltpu.VMEM((TCH,), I32),
                       pltpu.VMEM((TCH, 64), F32),
                       pltpu.VMEM((TCH, KK), F32),
                       pltpu.SemaphoreType.DMA),
    )(_target_gather_body)


def _target_gather_body(x_hbm, a1_hbm, a2_hbm, a3_hbm, tgt_hbm,
                   x_out, t1_out, t2_out, t3_out, idx_v, x_v, t_v, sem):
    wid = lax.axis_index("s") * 2 + lax.axis_index("c")
    base = wid * TCH
    pltpu.sync_copy(tgt_hbm.at[pl.ds(base, TCH)], idx_v)
    pltpu.async_copy(x_hbm.at[idx_v], x_v, sem).wait()
    pltpu.sync_copy(x_v, x_out.at[pl.ds(base, TCH)])
    for tab, out in ((a1_hbm, t1_out), (a2_hbm, t2_out), (a3_hbm, t3_out)):
        pltpu.async_copy(tab.at[idx_v], t_v, sem).wait()
        pltpu.sync_copy(t_v, out.at[pl.ds(base, TCH)])


def _softplus(x):
    return jnp.maximum(x, 0.0) + jnp.log(1.0 + jnp.exp(-jnp.abs(x)))


def _dist_body(x0_ref, t1_ref, t2_ref, t3_ref,
               w20_ref, b20_ref,
               g1_ref, nb1_ref, w11_ref, b11_ref, w21_ref, b21_ref,
               g2_ref, nb2_ref, w12_ref, b12_ref, w22_ref, b22_ref,
               g3_ref, nb3_ref, w13_ref, b13_ref, w23_ref, b23_ref,
               out_ref):
    h = _softplus(x0_ref[...])
    y = jnp.dot(h, w20_ref[...], preferred_element_type=F32) + b20_ref[...]
    for t_ref, g_ref, nb_ref, w1_ref, b1_ref, w2_ref, b2_ref in (
            (t1_ref, g1_ref, nb1_ref, w11_ref, b11_ref, w21_ref, b21_ref),
            (t2_ref, g2_ref, nb2_ref, w12_ref, b12_ref, w22_ref, b22_ref),
            (t3_ref, g3_ref, nb3_ref, w13_ref, b13_ref, w23_ref, b23_ref)):
        x = jax.nn.relu(t_ref[...])
        mu = jnp.mean(x, axis=-1, keepdims=True)
        va = jnp.var(x, axis=-1, keepdims=True)
        x = (x - mu) * lax.rsqrt(va + 1e-3) * g_ref[...] + nb_ref[...]
        h = _softplus(jnp.dot(x, w1_ref[...],
                              preferred_element_type=F32) + b1_ref[...])
        y = y + jnp.dot(h, w2_ref[...],
                        preferred_element_type=F32) + b2_ref[...]
    m = jnp.max(y, axis=-1, keepdims=True)
    e = jnp.exp(y - m)
    out_ref[...] = e / jnp.sum(e, axis=-1, keepdims=True)


def kernel(X, pos, target, conv_W, conv_b, norm_g, norm_b,
           dist_W1, dist_b1, dist_W2, dist_b2):
    n, P, C = X.shape
    DIMS = pos.shape[-1]
    Xc0 = jnp.pad(X.reshape(n, P * C), ((0, 0), (0, FP - P * C)))

    pos32 = pos.astype(I32)
    offset = jnp.array([1, 3, 3], I32)
    shifts = (jnp.arange(DIMS) * P).astype(I32)
    perms = []
    for i in range(DIMS):
        I = jnp.roll(pos32, shift=i, axis=-1) // offset
        key = jnp.sum(I << shifts, axis=-1)
        perms.append(jnp.argsort(key).astype(I32))

    S = n
    ar = jnp.arange(NTOT, dtype=I32)
    head = jnp.full((8,), S, I32)
    tail = jnp.full((PEXT - 8 - n,), S, I32)
    pexts = [jnp.concatenate([head, p, tail]) for p in perms]
    l1, r1, l2, r2, l3, r3 = _nbr_kernel()(*pexts)
    idx_list = [jnp.where((ar >= 1) & (ar < n), ar - 1, S),
                jnp.where(ar < n - 1, ar + 1, S),
                l1, r1, l2, r2, l3, r3]

    accs = []
    x = Xc0
    h0 = None
    for i in range(3):
        wcat = jnp.concatenate([conv_W[i][0], conv_W[i][1], conv_W[i][2]],
                               axis=-1)
        cb = conv_b[i].reshape(1, KK)
        if i == 0:
            wcat = jnp.pad(wcat, ((0, FP - P * C), (0, 0)))
            w10 = jnp.pad(dist_W1[0], ((0, FP - P * C), (0, 0)))
            wcat = jnp.concatenate([wcat, w10], axis=-1)
            a, bb, c, h0 = _proj(x, wcat, cb, db=dist_b1[0].reshape(1, 64))
        else:
            a, bb, c = _proj(x, wcat, cb, norm_g[i - 1].reshape(1, KK),
                             norm_b[i - 1].reshape(1, KK))
        acc = _gather_acc_kernel()(a, c, bb, *idx_list)
        accs.append(acc)
        x = acc

    xt0, t1, t2, t3 = _target_gather_kernel()(h0, accs[0], accs[1], accs[2],
                                              target)

    grid = T // BT
    full = lambda r, c: pl.BlockSpec((r, c), lambda m: (0, 0))
    blk = lambda c: pl.BlockSpec((BT, c), lambda m: (m, 0))
    args = [xt0, t1, t2, t3,
            dist_W2[0], dist_b2[0].reshape(1, -1)]
    specs = [blk(64), blk(KK), blk(KK), blk(KK),
             full(64, 2), full(1, 2)]
    for i in range(3):
        args += [norm_g[i].reshape(1, KK), norm_b[i].reshape(1, KK),
                 dist_W1[i + 1], dist_b1[i + 1].reshape(1, -1),
                 dist_W2[i + 1], dist_b2[i + 1].reshape(1, -1)]
        specs += [full(1, KK), full(1, KK),
                  full(KK, 64), full(1, 64), full(64, 2), full(1, 2)]
    out = pl.pallas_call(
        _dist_body, grid=(grid,),
        in_specs=specs,
        out_specs=blk(2),
        out_shape=jax.ShapeDtypeStruct((T, 2), F32),
    )(*args)
    return out


# speedup vs baseline: 5.8797x; 1.1743x over previous
import functools

import jax
import jax.numpy as jnp
from jax import lax
from jax.experimental import pallas as pl
from jax.experimental.pallas import tpu as pltpu
from jax.experimental.pallas import tpu_sc as plsc

F32 = jnp.float32
I32 = jnp.int32

N = 100000
NTOT = 100352
T = 16384
KK = 32
BM = 1024
NW = 32
CH = NTOT // NW
WIN = 392
NWIN = CH // WIN
TCH = T // NW
BT = 2048
FP = 136


def _proj_body(x_ref, w_ref, cb_ref, db_ref, w2h_ref,
               a_ref, bb_ref, c_ref, z_ref):
    m = pl.program_id(0)
    y = jnp.dot(x_ref[...], w_ref[...], preferred_element_type=F32)
    rows = m * BM + lax.broadcasted_iota(I32, (BM, 1), 0)
    msk = rows < N
    a_ref[...] = jnp.where(msk, y[:, 0:KK], 0.0)
    bb_ref[...] = jnp.where(msk, 4.0 * (y[:, KK:2 * KK] + cb_ref[...]), 0.0)
    c_ref[...] = jnp.where(msk, y[:, 2 * KK:3 * KK], 0.0)
    h = _softplus(y[:, 3 * KK:3 * KK + 64] + db_ref[...])
    z_ref[...] = jnp.dot(h, w2h_ref[...], preferred_element_type=F32)


def _proj_ln_body(x_ref, w_ref, cb_ref, g_ref, nb_ref,
                  w1h_ref, b1h_ref, w2h_ref,
                  a_ref, bb_ref, c_ref, z_ref):
    m = pl.program_id(0)
    x = jax.nn.relu(x_ref[...])
    mu = jnp.mean(x, axis=-1, keepdims=True)
    va = jnp.var(x, axis=-1, keepdims=True)
    x = (x - mu) * lax.rsqrt(va + 1e-3) * g_ref[...] + nb_ref[...]
    y = jnp.dot(x, w_ref[...], preferred_element_type=F32)
    rows = m * BM + lax.broadcasted_iota(I32, (BM, 1), 0)
    msk = rows < N
    a_ref[...] = jnp.where(msk, y[:, 0:KK], 0.0)
    bb_ref[...] = jnp.where(msk, 4.0 * (y[:, KK:2 * KK] + cb_ref[...]), 0.0)
    c_ref[...] = jnp.where(msk, y[:, 2 * KK:3 * KK], 0.0)
    h = _softplus(jnp.dot(x, w1h_ref[...], preferred_element_type=F32)
                  + b1h_ref[...])
    z_ref[...] = jnp.dot(h, w2h_ref[...], preferred_element_type=F32)


def _final_head_body(x_ref, g_ref, nb_ref, w1h_ref, b1h_ref, w2h_ref, z_ref):
    x = jax.nn.relu(x_ref[...])
    mu = jnp.mean(x, axis=-1, keepdims=True)
    va = jnp.var(x, axis=-1, keepdims=True)
    x = (x - mu) * lax.rsqrt(va + 1e-3) * g_ref[...] + nb_ref[...]
    h = _softplus(jnp.dot(x, w1h_ref[...], preferred_element_type=F32)
                  + b1h_ref[...])
    z_ref[...] = jnp.dot(h, w2h_ref[...], preferred_element_type=F32)


def _proj(x, wcat, cb, g=None, nb=None, db=None, w1h=None, b1h=None,
          w2h=None):
    cin = x.shape[1]
    grid = NTOT // BM
    outs = [jax.ShapeDtypeStruct((NTOT, KK), F32)] * 3 \
        + [jax.ShapeDtypeStruct((NTOT, 8), F32)]
    full = lambda r, c: pl.BlockSpec((r, c), lambda m: (0, 0))
    blk = lambda c: pl.BlockSpec((BM, c), lambda m: (m, 0))
    if g is None:
        return pl.pallas_call(
            _proj_body, grid=(grid,),
            in_specs=[blk(cin), full(cin, 3 * KK + 64), full(1, KK),
                      full(1, 64), full(64, 8)],
            out_specs=[blk(KK)] * 3 + [blk(8)],
            out_shape=outs,
        )(x, wcat, cb, db, w2h)
    return pl.pallas_call(
        _proj_ln_body, grid=(grid,),
        in_specs=[blk(cin), full(cin, 3 * KK), full(1, KK),
                  full(1, KK), full(1, KK),
                  full(KK, 64), full(1, 64), full(64, 8)],
        out_specs=[blk(KK)] * 3 + [blk(8)],
        out_shape=outs,
    )(x, wcat, cb, g, nb, w1h, b1h, w2h)


def _final_head(acc, g, nb, w1h, b1h, w2h):
    grid = NTOT // BM
    full = lambda r, c: pl.BlockSpec((r, c), lambda m: (0, 0))
    blk = lambda c: pl.BlockSpec((BM, c), lambda m: (m, 0))
    return pl.pallas_call(
        _final_head_body, grid=(grid,),
        in_specs=[blk(KK), full(1, KK), full(1, KK),
                  full(KK, 64), full(1, 64), full(64, 8)],
        out_specs=blk(8),
        out_shape=jax.ShapeDtypeStruct((NTOT, 8), F32),
    )(acc, g, nb, w1h, b1h, w2h)


@functools.lru_cache(maxsize=None)
def _sc_mesh():
    return plsc.VectorSubcoreMesh(core_axis_name="c", subcore_axis_name="s")


PEXT = NTOT + 16
CH6 = NTOT // 16
NPV6 = CH6 // 16
FCH = NTOT // 32


def _nbr_body(p1, p2, p3, l1, r1, l2, r2, l3, r3,
              ext, pv, lv, rv, sbuf, fbuf,
              shl1, shr1, shl2, shr2, shl3, shr3):
    cid = lax.axis_index("c")
    sid = lax.axis_index("s")
    base = sid * CH6
    svec = jnp.full((16,), N, I32)

    @pl.when(sid == 0)
    def _():
        for t in range((NTOT - N) // 16):
            sbuf[pl.ds(t * 16, 16)] = svec
        for sh in (shl1, shr1, shl2, shr2, shl3, shr3):
            pltpu.sync_copy(sbuf, sh.at[pl.ds(N, NTOT - N)])

    iota = lax.iota(I32, 16)
    for p_hbm, sh_lo, sh_ro in ((p1, shl1, shr1), (p2, shl2, shr2),
                                (p3, shl3, shr3)):
        pltpu.sync_copy(p_hbm.at[pl.ds(base, CH6 + 16)], ext)
        pltpu.sync_copy(p_hbm.at[pl.ds(base + 8, CH6)], pv)

        def body(t, _):
            k0 = t * 16
            lv[pl.ds(k0, 16)] = plsc.load_gather(ext, [iota + (k0 + 7)])
            rv[pl.ds(k0, 16)] = plsc.load_gather(ext, [iota + (k0 + 9)])
            return 0

        lax.fori_loop(0, NPV6, body, 0)
        pltpu.sync_copy(lv, sh_lo.at[pv])
        pltpu.sync_copy(rv, sh_ro.at[pv])

    plsc.subcore_barrier()
    fbase = cid * (NTOT // 2) + sid * FCH
    for sh, out in ((shl1, l1), (shr1, r1), (shl2, l2), (shr2, r2),
                    (shl3, l3), (shr3, r3)):
        pltpu.sync_copy(sh.at[pl.ds(fbase, FCH)], fbuf)
        pltpu.sync_copy(fbuf, out.at[pl.ds(fbase, FCH)])


@functools.lru_cache(maxsize=None)
def _nbr_kernel():
    return functools.partial(
        pl.kernel,
        out_type=tuple(jax.ShapeDtypeStruct((NTOT,), I32) for _ in range(6)),
        mesh=_sc_mesh(),
        compiler_params=pltpu.CompilerParams(use_tc_tiling_on_sc=False,
                                             needs_layout_passes=False),
        scratch_types=(pltpu.VMEM((CH6 + 16,), I32),
                       pltpu.VMEM((CH6,), I32),
                       pltpu.VMEM((CH6,), I32),
                       pltpu.VMEM((CH6,), I32),
                       pltpu.VMEM((NTOT - N,), I32),
                       pltpu.VMEM((FCH,), I32))
        + tuple(pltpu.VMEM_SHARED((NTOT,), I32) for _ in range(6)),
    )(_nbr_body)


@functools.lru_cache(maxsize=None)
def _gather_acc_kernel():
    return functools.partial(
        pl.kernel,
        out_type=jax.ShapeDtypeStruct((NTOT, KK), F32),
        mesh=_sc_mesh(),
        compiler_params=pltpu.CompilerParams(use_tc_tiling_on_sc=False),
        scratch_types=(
            [pltpu.VMEM((WIN,), I32) for _ in range(8)]
            + [pltpu.VMEM((WIN, KK), F32) for _ in range(8)]
            + [pltpu.VMEM((WIN, KK), F32), pltpu.SemaphoreType.DMA]
        ),
    )(_gather_acc_body)


def _gather_acc_body(a_hbm, c_hbm, bb_hbm,
                x0_hbm, x1_hbm, x2_hbm, x3_hbm,
                x4_hbm, x5_hbm, x6_hbm, x7_hbm, out_hbm,
                i0, i1, i2, i3, i4, i5, i6, i7,
                g0, g1, g2, g3, g4, g5, g6, g7, accv, sem):
    wid = lax.axis_index("s") * 2 + lax.axis_index("c")
    idx_hbms = (x0_hbm, x1_hbm, x2_hbm, x3_hbm, x4_hbm, x5_hbm, x6_hbm, x7_hbm)
    ivs = (i0, i1, i2, i3, i4, i5, i6, i7)
    gvs = (g0, g1, g2, g3, g4, g5, g6, g7)
    for k in range(NWIN):
        base = wid * CH + k * WIN
        pltpu.sync_copy(bb_hbm.at[pl.ds(base, WIN)], accv)
        for t in range(8):
            pltpu.sync_copy(idx_hbms[t].at[pl.ds(base, WIN)], ivs[t])
        cps = []
        for t in range(8):
            tab = a_hbm if t % 2 == 0 else c_hbm
            cps.append(pltpu.async_copy(tab.at[ivs[t]], gvs[t], sem))
        for cp in cps:
            cp.wait()

        def row_body(r, _):
            for h in (0, 16):
                v = accv[r, pl.ds(h, 16)]
                for gv in gvs:
                    v = v + gv[r, pl.ds(h, 16)]
                accv[r, pl.ds(h, 16)] = v
            return 0

        lax.fori_loop(0, WIN, row_body, 0)
        pltpu.sync_copy(accv, out_hbm.at[pl.ds(base, WIN)])


@functools.lru_cache(maxsize=None)
def _target_gather_kernel():
    return functools.partial(
        pl.kernel,
        out_type=tuple(jax.ShapeDtypeStruct((T, 8), F32) for _ in range(4)),
        mesh=_sc_mesh(),
        compiler_params=pltpu.CompilerParams(use_tc_tiling_on_sc=False),
        scratch_types=(pltpu.VMEM((TCH,), I32),
                       pltpu.VMEM((TCH, 8), F32),
                       pltpu.VMEM((TCH, 8), F32),
                       pltpu.VMEM((TCH, 8), F32),
                       pltpu.VMEM((TCH, 8), F32),
                       pltpu.SemaphoreType.DMA),
    )(_target_gather_body)


def _target_gather_body(x_hbm, a1_hbm, a2_hbm, a3_hbm, tgt_hbm,
                        x_out, t1_out, t2_out, t3_out,
                        idx_v, x_v, t1_v, t2_v, t3_v, sem):
    wid = lax.axis_index("s") * 2 + lax.axis_index("c")
    base = wid * TCH
    pltpu.sync_copy(tgt_hbm.at[pl.ds(base, TCH)], idx_v)
    cps = [pltpu.async_copy(x_hbm.at[idx_v], x_v, sem),
           pltpu.async_copy(a1_hbm.at[idx_v], t1_v, sem),
           pltpu.async_copy(a2_hbm.at[idx_v], t2_v, sem),
           pltpu.async_copy(a3_hbm.at[idx_v], t3_v, sem)]
    for cp in cps:
        cp.wait()
    pltpu.sync_copy(x_v, x_out.at[pl.ds(base, TCH)])
    pltpu.sync_copy(t1_v, t1_out.at[pl.ds(base, TCH)])
    pltpu.sync_copy(t2_v, t2_out.at[pl.ds(base, TCH)])
    pltpu.sync_copy(t3_v, t3_out.at[pl.ds(base, TCH)])


def _softplus(x):
    return jnp.maximum(x, 0.0) + jnp.log(1.0 + jnp.exp(-jnp.abs(x)))


def _dist_body(z0_ref, z1_ref, z2_ref, z3_ref, bsum_ref, out_ref):
    y = (z0_ref[...] + z1_ref[...] + z2_ref[...] + z3_ref[...]
         + bsum_ref[...])[:, 0:2]
    m = jnp.max(y, axis=-1, keepdims=True)
    e = jnp.exp(y - m)
    out_ref[...] = e / jnp.sum(e, axis=-1, keepdims=True)


def kernel(X, pos, target, conv_W, conv_b, norm_g, norm_b,
           dist_W1, dist_b1, dist_W2, dist_b2):
    n, P, C = X.shape
    DIMS = pos.shape[-1]
    Xc0 = jnp.pad(X.reshape(n, P * C), ((0, 0), (0, FP - P * C)))

    pos32 = pos.astype(I32)
    offset = jnp.array([1, 3, 3], I32)
    shifts = (jnp.arange(DIMS) * P).astype(I32)
    perms = []
    for i in range(DIMS):
        I = jnp.roll(pos32, shift=i, axis=-1) // offset
        key = jnp.sum(I << shifts, axis=-1)
        perms.append(jnp.argsort(key).astype(I32))

    S = n
    ar = jnp.arange(NTOT, dtype=I32)
    head = jnp.full((8,), S, I32)
    tail = jnp.full((PEXT - 8 - n,), S, I32)
    pexts = [jnp.concatenate([head, p, tail]) for p in perms]
    l1, r1, l2, r2, l3, r3 = _nbr_kernel()(*pexts)
    idx_list = [jnp.where((ar >= 1) & (ar < n), ar - 1, S),
                jnp.where(ar < n - 1, ar + 1, S),
                l1, r1, l2, r2, l3, r3]

    w2p = [jnp.pad(w, ((0, 0), (0, 6))) for w in dist_W2]
    zs = []
    x = Xc0
    for i in range(3):
        wcat = jnp.concatenate([conv_W[i][0], conv_W[i][1], conv_W[i][2]],
                               axis=-1)
        cb = conv_b[i].reshape(1, KK)
        if i == 0:
            wcat = jnp.pad(wcat, ((0, FP - P * C), (0, 0)))
            w10 = jnp.pad(dist_W1[0], ((0, FP - P * C), (0, 0)))
            wcat = jnp.concatenate([wcat, w10], axis=-1)
            a, bb, c, z = _proj(x, wcat, cb, db=dist_b1[0].reshape(1, 64),
                                w2h=w2p[0])
        else:
            a, bb, c, z = _proj(x, wcat, cb, norm_g[i - 1].reshape(1, KK),
                                norm_b[i - 1].reshape(1, KK),
                                w1h=dist_W1[i], b1h=dist_b1[i].reshape(1, 64),
                                w2h=w2p[i])
        zs.append(z)
        acc = _gather_acc_kernel()(a, c, bb, *idx_list)
        x = acc
    zs.append(_final_head(x, norm_g[2].reshape(1, KK),
                          norm_b[2].reshape(1, KK), dist_W1[3],
                          dist_b1[3].reshape(1, 64), w2p[3]))

    z0g, z1g, z2g, z3g = _target_gather_kernel()(zs[0], zs[1], zs[2], zs[3],
                                                 target)
    bsum = (dist_b2[0] + dist_b2[1] + dist_b2[2] + dist_b2[3])
    bsum = jnp.pad(bsum, (0, 6)).reshape(1, 8)
    grid = T // BT
    blk = lambda c: pl.BlockSpec((BT, c), lambda m: (m, 0))
    out = pl.pallas_call(
        _dist_body, grid=(grid,),
        in_specs=[blk(8)] * 4 + [pl.BlockSpec((1, 8), lambda m: (0, 0))],
        out_specs=blk(2),
        out_shape=jax.ShapeDtypeStruct((T, 2), F32),
    )(z0g, z1g, z2g, z3g, bsum)
    return out


# speedup vs baseline: 6.1537x; 1.0466x over previous
import functools

import jax
import jax.numpy as jnp
from jax import lax
from jax.experimental import pallas as pl
from jax.experimental.pallas import tpu as pltpu
from jax.experimental.pallas import tpu_sc as plsc

F32 = jnp.float32
I32 = jnp.int32

N = 100000
NTOT = 100352
T = 16384
KK = 32
BM = 1024
NW = 32
CH = NTOT // NW
WIN = 392
NWIN = CH // WIN
TCH = T // NW
BT = 2048
FP = 136


def _proj_body(x_ref, w_ref, cb_ref, db_ref, w2h_ref,
               a_ref, bb_ref, c_ref, z_ref):
    m = pl.program_id(0)
    y = jnp.dot(x_ref[...], w_ref[...], preferred_element_type=F32)
    rows = m * BM + lax.broadcasted_iota(I32, (BM, 1), 0)
    msk = rows < N
    a_ref[...] = jnp.where(msk, y[:, 0:KK], 0.0)
    bb_ref[...] = jnp.where(msk, 4.0 * (y[:, KK:2 * KK] + cb_ref[...]), 0.0)
    c_ref[...] = jnp.where(msk, y[:, 2 * KK:3 * KK], 0.0)
    h = _softplus(y[:, 3 * KK:3 * KK + 64] + db_ref[...])
    z_ref[...] = jnp.dot(h, w2h_ref[...], preferred_element_type=F32)


def _proj_ln_body(x_ref, w_ref, cb_ref, g_ref, nb_ref,
                  w1h_ref, b1h_ref, w2h_ref,
                  a_ref, bb_ref, c_ref, z_ref):
    m = pl.program_id(0)
    x = jax.nn.relu(x_ref[...])
    mu = jnp.mean(x, axis=-1, keepdims=True)
    va = jnp.var(x, axis=-1, keepdims=True)
    x = (x - mu) * lax.rsqrt(va + 1e-3) * g_ref[...] + nb_ref[...]
    y = jnp.dot(x, w_ref[...], preferred_element_type=F32)
    rows = m * BM + lax.broadcasted_iota(I32, (BM, 1), 0)
    msk = rows < N
    a_ref[...] = jnp.where(msk, y[:, 0:KK], 0.0)
    bb_ref[...] = jnp.where(msk, 4.0 * (y[:, KK:2 * KK] + cb_ref[...]), 0.0)
    c_ref[...] = jnp.where(msk, y[:, 2 * KK:3 * KK], 0.0)
    h = _softplus(jnp.dot(x, w1h_ref[...], preferred_element_type=F32)
                  + b1h_ref[...])
    z_ref[...] = jnp.dot(h, w2h_ref[...], preferred_element_type=F32)


def _final_head_body(x_ref, g_ref, nb_ref, w1h_ref, b1h_ref, w2h_ref, z_ref):
    x = jax.nn.relu(x_ref[...])
    mu = jnp.mean(x, axis=-1, keepdims=True)
    va = jnp.var(x, axis=-1, keepdims=True)
    x = (x - mu) * lax.rsqrt(va + 1e-3) * g_ref[...] + nb_ref[...]
    h = _softplus(jnp.dot(x, w1h_ref[...], preferred_element_type=F32)
                  + b1h_ref[...])
    z_ref[...] = jnp.dot(h, w2h_ref[...], preferred_element_type=F32)


def _proj(x, wcat, cb, g=None, nb=None, db=None, w1h=None, b1h=None,
          w2h=None):
    cin = x.shape[1]
    grid = NTOT // BM
    outs = [jax.ShapeDtypeStruct((NTOT, KK), F32)] * 3 \
        + [jax.ShapeDtypeStruct((NTOT, 8), F32)]
    full = lambda r, c: pl.BlockSpec((r, c), lambda m: (0, 0))
    blk = lambda c: pl.BlockSpec((BM, c), lambda m: (m, 0))
    if g is None:
        return pl.pallas_call(
            _proj_body, grid=(grid,),
            in_specs=[blk(cin), full(cin, 3 * KK + 64), full(1, KK),
                      full(1, 64), full(64, 8)],
            out_specs=[blk(KK)] * 3 + [blk(8)],
            out_shape=outs,
        )(x, wcat, cb, db, w2h)
    return pl.pallas_call(
        _proj_ln_body, grid=(grid,),
        in_specs=[blk(cin), full(cin, 3 * KK), full(1, KK),
                  full(1, KK), full(1, KK),
                  full(KK, 64), full(1, 64), full(64, 8)],
        out_specs=[blk(KK)] * 3 + [blk(8)],
        out_shape=outs,
    )(x, wcat, cb, g, nb, w1h, b1h, w2h)


def _final_head(acc, g, nb, w1h, b1h, w2h):
    grid = NTOT // BM
    full = lambda r, c: pl.BlockSpec((r, c), lambda m: (0, 0))
    blk = lambda c: pl.BlockSpec((BM, c), lambda m: (m, 0))
    return pl.pallas_call(
        _final_head_body, grid=(grid,),
        in_specs=[blk(KK), full(1, KK), full(1, KK),
                  full(KK, 64), full(1, 64), full(64, 8)],
        out_specs=blk(8),
        out_shape=jax.ShapeDtypeStruct((NTOT, 8), F32),
    )(acc, g, nb, w1h, b1h, w2h)


@functools.lru_cache(maxsize=None)
def _sc_mesh():
    return plsc.VectorSubcoreMesh(core_axis_name="c", subcore_axis_name="s")


PEXT = NTOT + 16
CH6 = NTOT // 16
NPV6 = CH6 // 16
FCH = NTOT // 32


def _nbr_body(p1, p2, p3, l1, r1, l2, r2, l3, r3,
              ext, pv, lv, rv, sbuf, fbuf,
              shl1, shr1, shl2, shr2, shl3, shr3):
    cid = lax.axis_index("c")
    sid = lax.axis_index("s")
    base = sid * CH6
    svec = jnp.full((16,), N, I32)

    @pl.when(sid == 0)
    def _():
        for t in range((NTOT - N) // 16):
            sbuf[pl.ds(t * 16, 16)] = svec
        for sh in (shl1, shr1, shl2, shr2, shl3, shr3):
            pltpu.sync_copy(sbuf, sh.at[pl.ds(N, NTOT - N)])

    iota = lax.iota(I32, 16)
    for p_hbm, sh_lo, sh_ro in ((p1, shl1, shr1), (p2, shl2, shr2),
                                (p3, shl3, shr3)):
        pltpu.sync_copy(p_hbm.at[pl.ds(base, CH6 + 16)], ext)
        pltpu.sync_copy(p_hbm.at[pl.ds(base + 8, CH6)], pv)

        def body(t, _):
            k0 = t * 16
            lv[pl.ds(k0, 16)] = plsc.load_gather(ext, [iota + (k0 + 7)])
            rv[pl.ds(k0, 16)] = plsc.load_gather(ext, [iota + (k0 + 9)])
            return 0

        lax.fori_loop(0, NPV6, body, 0)
        pltpu.sync_copy(lv, sh_lo.at[pv])
        pltpu.sync_copy(rv, sh_ro.at[pv])

    plsc.subcore_barrier()
    fbase = cid * (NTOT // 2) + sid * FCH
    for sh, out in ((shl1, l1), (shr1, r1), (shl2, l2), (shr2, r2),
                    (shl3, l3), (shr3, r3)):
        pltpu.sync_copy(sh.at[pl.ds(fbase, FCH)], fbuf)
        pltpu.sync_copy(fbuf, out.at[pl.ds(fbase, FCH)])


@functools.lru_cache(maxsize=None)
def _nbr_kernel():
    return functools.partial(
        pl.kernel,
        out_type=tuple(jax.ShapeDtypeStruct((NTOT,), I32) for _ in range(6)),
        mesh=_sc_mesh(),
        compiler_params=pltpu.CompilerParams(use_tc_tiling_on_sc=False,
                                             needs_layout_passes=False),
        scratch_types=(pltpu.VMEM((CH6 + 16,), I32),
                       pltpu.VMEM((CH6,), I32),
                       pltpu.VMEM((CH6,), I32),
                       pltpu.VMEM((CH6,), I32),
                       pltpu.VMEM((NTOT - N,), I32),
                       pltpu.VMEM((FCH,), I32))
        + tuple(pltpu.VMEM_SHARED((NTOT,), I32) for _ in range(6)),
    )(_nbr_body)


@functools.lru_cache(maxsize=None)
def _gather_acc_kernel():
    return functools.partial(
        pl.kernel,
        out_type=jax.ShapeDtypeStruct((NTOT, KK), F32),
        mesh=_sc_mesh(),
        compiler_params=pltpu.CompilerParams(use_tc_tiling_on_sc=False),
        scratch_types=(
            [pltpu.VMEM((WIN,), I32) for _ in range(8)]
            + [pltpu.VMEM((WIN, KK), F32) for _ in range(8)]
            + [pltpu.VMEM((WIN, KK), F32), pltpu.SemaphoreType.DMA]
        ),
    )(_gather_acc_body)


def _gather_acc_body(a_hbm, c_hbm, bb_hbm,
                x0_hbm, x1_hbm, x2_hbm, x3_hbm,
                x4_hbm, x5_hbm, x6_hbm, x7_hbm, out_hbm,
                i0, i1, i2, i3, i4, i5, i6, i7,
                g0, g1, g2, g3, g4, g5, g6, g7, accv, sem):
    wid = lax.axis_index("s") * 2 + lax.axis_index("c")
    idx_hbms = (x0_hbm, x1_hbm, x2_hbm, x3_hbm, x4_hbm, x5_hbm, x6_hbm, x7_hbm)
    ivs = (i0, i1, i2, i3, i4, i5, i6, i7)
    gvs = (g0, g1, g2, g3, g4, g5, g6, g7)
    for k in range(NWIN):
        base = wid * CH + k * WIN
        pltpu.sync_copy(bb_hbm.at[pl.ds(base, WIN)], accv)
        for t in range(8):
            pltpu.sync_copy(idx_hbms[t].at[pl.ds(base, WIN)], ivs[t])
        cps = []
        for t in range(8):
            tab = a_hbm if t % 2 == 0 else c_hbm
            cps.append(pltpu.async_copy(tab.at[ivs[t]], gvs[t], sem))
        for cp in cps:
            cp.wait()

        def row_body(r, _):
            for h in (0, 16):
                v = accv[r, pl.ds(h, 16)]
                for gv in gvs:
                    v = v + gv[r, pl.ds(h, 16)]
                accv[r, pl.ds(h, 16)] = v
            return 0

        lax.fori_loop(0, WIN, row_body, 0)
        pltpu.sync_copy(accv, out_hbm.at[pl.ds(base, WIN)])


@functools.lru_cache(maxsize=None)
def _target_gather_kernel():
    return functools.partial(
        pl.kernel,
        out_type=tuple(jax.ShapeDtypeStruct((T, 8), F32) for _ in range(4)),
        mesh=_sc_mesh(),
        compiler_params=pltpu.CompilerParams(use_tc_tiling_on_sc=False),
        scratch_types=(pltpu.VMEM((TCH,), I32),
                       pltpu.VMEM((TCH, 8), F32),
                       pltpu.VMEM((TCH, 8), F32),
                       pltpu.VMEM((TCH, 8), F32),
                       pltpu.VMEM((TCH, 8), F32),
                       pltpu.SemaphoreType.DMA),
    )(_target_gather_body)


def _target_gather_body(x_hbm, a1_hbm, a2_hbm, a3_hbm, tgt_hbm,
                        x_out, t1_out, t2_out, t3_out,
                        idx_v, x_v, t1_v, t2_v, t3_v, sem):
    wid = lax.axis_index("s") * 2 + lax.axis_index("c")
    base = wid * TCH
    pltpu.sync_copy(tgt_hbm.at[pl.ds(base, TCH)], idx_v)
    cps = [pltpu.async_copy(x_hbm.at[idx_v], x_v, sem),
           pltpu.async_copy(a1_hbm.at[idx_v], t1_v, sem),
           pltpu.async_copy(a2_hbm.at[idx_v], t2_v, sem),
           pltpu.async_copy(a3_hbm.at[idx_v], t3_v, sem)]
    for cp in cps:
        cp.wait()
    pltpu.sync_copy(x_v, x_out.at[pl.ds(base, TCH)])
    pltpu.sync_copy(t1_v, t1_out.at[pl.ds(base, TCH)])
    pltpu.sync_copy(t2_v, t2_out.at[pl.ds(base, TCH)])
    pltpu.sync_copy(t3_v, t3_out.at[pl.ds(base, TCH)])


def _softplus(x):
    return jnp.maximum(x, 0.0) + jnp.log(1.0 + jnp.exp(-jnp.abs(x)))


def _dist_body(z0_ref, z1_ref, z2_ref, z3_ref, bsum_ref, out_ref):
    y = (z0_ref[...] + z1_ref[...] + z2_ref[...] + z3_ref[...]
         + bsum_ref[...])[:, 0:2]
    m = jnp.max(y, axis=-1, keepdims=True)
    e = jnp.exp(y - m)
    out_ref[...] = e / jnp.sum(e, axis=-1, keepdims=True)


def kernel(X, pos, target, conv_W, conv_b, norm_g, norm_b,
           dist_W1, dist_b1, dist_W2, dist_b2):
    n, P, C = X.shape
    DIMS = pos.shape[-1]
    Xc0 = X.reshape(n, P * C)

    pos32 = pos.astype(I32)
    offset = jnp.array([1, 3, 3], I32)
    shifts = (jnp.arange(DIMS) * P).astype(I32)
    perms = []
    for i in range(DIMS):
        I = jnp.roll(pos32, shift=i, axis=-1) // offset
        key = jnp.sum(I << shifts, axis=-1)
        perms.append(jnp.argsort(key).astype(I32))

    S = n
    ar = jnp.arange(NTOT, dtype=I32)
    head = jnp.full((8,), S, I32)
    tail = jnp.full((PEXT - 8 - n,), S, I32)
    pexts = [jnp.concatenate([head, p, tail]) for p in perms]
    l1, r1, l2, r2, l3, r3 = _nbr_kernel()(*pexts)
    idx_list = [jnp.where((ar >= 1) & (ar < n), ar - 1, S),
                jnp.where(ar < n - 1, ar + 1, S),
                l1, r1, l2, r2, l3, r3]

    w2p = [jnp.pad(w, ((0, 0), (0, 6))) for w in dist_W2]
    zs = []
    x = Xc0
    for i in range(3):
        wcat = jnp.concatenate([conv_W[i][0], conv_W[i][1], conv_W[i][2]],
                               axis=-1)
        cb = conv_b[i].reshape(1, KK)
        if i == 0:
            wcat = jnp.concatenate([wcat, dist_W1[0]], axis=-1)
            a, bb, c, z = _proj(x, wcat, cb, db=dist_b1[0].reshape(1, 64),
                                w2h=w2p[0])
        else:
            a, bb, c, z = _proj(x, wcat, cb, norm_g[i - 1].reshape(1, KK),
                                norm_b[i - 1].reshape(1, KK),
                                w1h=dist_W1[i], b1h=dist_b1[i].reshape(1, 64),
                                w2h=w2p[i])
        zs.append(z)
        acc = _gather_acc_kernel()(a, c, bb, *idx_list)
        x = acc
    zs.append(_final_head(x, norm_g[2].reshape(1, KK),
                          norm_b[2].reshape(1, KK), dist_W1[3],
                          dist_b1[3].reshape(1, 64), w2p[3]))

    z0g, z1g, z2g, z3g = _target_gather_kernel()(zs[0], zs[1], zs[2], zs[3],
                                                 target)
    bsum = (dist_b2[0] + dist_b2[1] + dist_b2[2] + dist_b2[3])
    bsum = jnp.pad(bsum, (0, 6)).reshape(1, 8)
    grid = T // BT
    blk = lambda c: pl.BlockSpec((BT, c), lambda m: (m, 0))
    out = pl.pallas_call(
        _dist_body, grid=(grid,),
        in_specs=[blk(8)] * 4 + [pl.BlockSpec((1, 8), lambda m: (0, 0))],
        out_specs=blk(2),
        out_shape=jax.ShapeDtypeStruct((T, 2), F32),
    )(z0g, z1g, z2g, z3g, bsum)
    return out


# speedup vs baseline: 6.2882x; 1.0219x over previous
import functools

import jax
import jax.numpy as jnp
from jax import lax
from jax.experimental import pallas as pl
from jax.experimental.pallas import tpu as pltpu
from jax.experimental.pallas import tpu_sc as plsc

F32 = jnp.float32
I32 = jnp.int32

N = 100000
NTOT = 100352
T = 16384
KK = 32
BM = 1024
NW = 32
CH = NTOT // NW
WIN = 392
NWIN = CH // WIN
TCH = T // NW
BT = 2048
FP = 136


def _proj_body(x_ref, w_ref, cb_ref, db_ref, w2h_ref,
               a_ref, bb_ref, c_ref, z_ref):
    m = pl.program_id(0)
    y = jnp.dot(x_ref[...], w_ref[...], preferred_element_type=F32)
    rows = m * BM + lax.broadcasted_iota(I32, (BM, 1), 0)
    msk = rows < N
    a_ref[...] = jnp.where(msk, y[:, 0:KK], 0.0)
    bb_ref[...] = jnp.where(msk, 4.0 * (y[:, KK:2 * KK] + cb_ref[...]), 0.0)
    c_ref[...] = jnp.where(msk, y[:, 2 * KK:3 * KK], 0.0)
    h = _softplus(y[:, 3 * KK:3 * KK + 64] + db_ref[...])
    z_ref[...] = jnp.dot(h, w2h_ref[...], preferred_element_type=F32)


def _proj_ln_body(x_ref, w_ref, cb_ref, g_ref, nb_ref,
                  w1h_ref, b1h_ref, w2h_ref,
                  a_ref, bb_ref, c_ref, z_ref):
    m = pl.program_id(0)
    x = jax.nn.relu(x_ref[...])
    mu = jnp.mean(x, axis=-1, keepdims=True)
    va = jnp.var(x, axis=-1, keepdims=True)
    x = (x - mu) * lax.rsqrt(va + 1e-3) * g_ref[...] + nb_ref[...]
    y = jnp.dot(x, w_ref[...], preferred_element_type=F32)
    rows = m * BM + lax.broadcasted_iota(I32, (BM, 1), 0)
    msk = rows < N
    a_ref[...] = jnp.where(msk, y[:, 0:KK], 0.0)
    bb_ref[...] = jnp.where(msk, 4.0 * (y[:, KK:2 * KK] + cb_ref[...]), 0.0)
    c_ref[...] = jnp.where(msk, y[:, 2 * KK:3 * KK], 0.0)
    h = _softplus(jnp.dot(x, w1h_ref[...], preferred_element_type=F32)
                  + b1h_ref[...])
    z_ref[...] = jnp.dot(h, w2h_ref[...], preferred_element_type=F32)


def _final_head_body(x_ref, g_ref, nb_ref, w1h_ref, b1h_ref, w2h_ref, z_ref):
    x = jax.nn.relu(x_ref[...])
    mu = jnp.mean(x, axis=-1, keepdims=True)
    va = jnp.var(x, axis=-1, keepdims=True)
    x = (x - mu) * lax.rsqrt(va + 1e-3) * g_ref[...] + nb_ref[...]
    h = _softplus(jnp.dot(x, w1h_ref[...], preferred_element_type=F32)
                  + b1h_ref[...])
    z_ref[...] = jnp.dot(h, w2h_ref[...], preferred_element_type=F32)


def _proj(x, wcat, cb, g=None, nb=None, db=None, w1h=None, b1h=None,
          w2h=None):
    cin = x.shape[1]
    grid = NTOT // BM
    outs = [jax.ShapeDtypeStruct((NTOT, KK), F32)] * 3 \
        + [jax.ShapeDtypeStruct((NTOT, 8), F32)]
    full = lambda r, c: pl.BlockSpec((r, c), lambda m: (0, 0))
    blk = lambda c: pl.BlockSpec((BM, c), lambda m: (m, 0))
    if g is None:
        return pl.pallas_call(
            _proj_body, grid=(grid,),
            in_specs=[blk(cin), full(cin, 3 * KK + 64), full(1, KK),
                      full(1, 64), full(64, 8)],
            out_specs=[blk(KK)] * 3 + [blk(8)],
            out_shape=outs,
        )(x, wcat, cb, db, w2h)
    return pl.pallas_call(
        _proj_ln_body, grid=(grid,),
        in_specs=[blk(cin), full(cin, 3 * KK), full(1, KK),
                  full(1, KK), full(1, KK),
                  full(KK, 64), full(1, 64), full(64, 8)],
        out_specs=[blk(KK)] * 3 + [blk(8)],
        out_shape=outs,
    )(x, wcat, cb, g, nb, w1h, b1h, w2h)


def _final_head(acc, g, nb, w1h, b1h, w2h):
    grid = NTOT // BM
    full = lambda r, c: pl.BlockSpec((r, c), lambda m: (0, 0))
    blk = lambda c: pl.BlockSpec((BM, c), lambda m: (m, 0))
    return pl.pallas_call(
        _final_head_body, grid=(grid,),
        in_specs=[blk(KK), full(1, KK), full(1, KK),
                  full(KK, 64), full(1, 64), full(64, 8)],
        out_specs=blk(8),
        out_shape=jax.ShapeDtypeStruct((NTOT, 8), F32),
    )(acc, g, nb, w1h, b1h, w2h)


@functools.lru_cache(maxsize=None)
def _sc_mesh():
    return plsc.VectorSubcoreMesh(core_axis_name="c", subcore_axis_name="s")


PEXT = NTOT + 16
CH6 = NTOT // 16
NPV6 = CH6 // 16
FCH = NTOT // 32


def _nbr_body(p1, p2, p3, l1, r1, l2, r2, l3, r3,
              ext, pv, lv, rv, sbuf, fbuf,
              shl1, shr1, shl2, shr2, shl3, shr3):
    cid = lax.axis_index("c")
    sid = lax.axis_index("s")
    base = sid * CH6
    svec = jnp.full((16,), N, I32)

    @pl.when(sid == 0)
    def _():
        for t in range((NTOT - N) // 16):
            sbuf[pl.ds(t * 16, 16)] = svec
        for sh in (shl1, shr1, shl2, shr2, shl3, shr3):
            pltpu.sync_copy(sbuf, sh.at[pl.ds(N, NTOT - N)])

    iota = lax.iota(I32, 16)
    for p_hbm, sh_lo, sh_ro in ((p1, shl1, shr1), (p2, shl2, shr2),
                                (p3, shl3, shr3)):
        pltpu.sync_copy(p_hbm.at[pl.ds(base, CH6 + 16)], ext)
        pltpu.sync_copy(p_hbm.at[pl.ds(base + 8, CH6)], pv)

        def body(t, _):
            k0 = t * 16
            lv[pl.ds(k0, 16)] = plsc.load_gather(ext, [iota + (k0 + 7)])
            rv[pl.ds(k0, 16)] = plsc.load_gather(ext, [iota + (k0 + 9)])
            return 0

        lax.fori_loop(0, NPV6, body, 0)
        pltpu.sync_copy(lv, sh_lo.at[pv])
        pltpu.sync_copy(rv, sh_ro.at[pv])

    plsc.subcore_barrier()
    fbase = cid * (NTOT // 2) + sid * FCH
    for sh, out in ((shl1, l1), (shr1, r1), (shl2, l2), (shr2, r2),
                    (shl3, l3), (shr3, r3)):
        pltpu.sync_copy(sh.at[pl.ds(fbase, FCH)], fbuf)
        pltpu.sync_copy(fbuf, out.at[pl.ds(fbase, FCH)])


@functools.lru_cache(maxsize=None)
def _nbr_kernel():
    return functools.partial(
        pl.kernel,
        out_type=tuple(jax.ShapeDtypeStruct((NTOT,), I32) for _ in range(6)),
        mesh=_sc_mesh(),
        compiler_params=pltpu.CompilerParams(use_tc_tiling_on_sc=False,
                                             needs_layout_passes=False),
        scratch_types=(pltpu.VMEM((CH6 + 16,), I32),
                       pltpu.VMEM((CH6,), I32),
                       pltpu.VMEM((CH6,), I32),
                       pltpu.VMEM((CH6,), I32),
                       pltpu.VMEM((NTOT - N,), I32),
                       pltpu.VMEM((FCH,), I32))
        + tuple(pltpu.VMEM_SHARED((NTOT,), I32) for _ in range(6)),
    )(_nbr_body)


@functools.lru_cache(maxsize=None)
def _gather_acc_kernel():
    return functools.partial(
        pl.kernel,
        out_type=jax.ShapeDtypeStruct((NTOT, KK), F32),
        mesh=_sc_mesh(),
        compiler_params=pltpu.CompilerParams(use_tc_tiling_on_sc=False),
        scratch_types=(
            [pltpu.VMEM((8 * WIN,), I32)]
            + [pltpu.VMEM((WIN, KK), F32) for _ in range(8)]
            + [pltpu.VMEM((WIN, KK), F32), pltpu.SemaphoreType.DMA]
        ),
    )(_gather_acc_body)


def _gather_acc_body(a_hbm, c_hbm, bb_hbm, idxp_hbm, out_hbm,
                     idxv, g0, g1, g2, g3, g4, g5, g6, g7, accv, sem):
    wid = lax.axis_index("s") * 2 + lax.axis_index("c")
    gvs = (g0, g1, g2, g3, g4, g5, g6, g7)
    for k in range(NWIN):
        base = wid * CH + k * WIN
        pltpu.sync_copy(bb_hbm.at[pl.ds(base, WIN)], accv)
        pltpu.sync_copy(idxp_hbm.at[pl.ds(base * 8, 8 * WIN)], idxv)
        cps = []
        for t in range(8):
            tab = a_hbm if t % 2 == 0 else c_hbm
            cps.append(pltpu.async_copy(tab.at[idxv.at[pl.ds(t * WIN, WIN)]],
                                        gvs[t], sem))
        for cp in cps:
            cp.wait()

        def row_body(r, _):
            for h in (0, 16):
                v = accv[r, pl.ds(h, 16)]
                for gv in gvs:
                    v = v + gv[r, pl.ds(h, 16)]
                accv[r, pl.ds(h, 16)] = v
            return 0

        lax.fori_loop(0, WIN, row_body, 0)
        pltpu.sync_copy(accv, out_hbm.at[pl.ds(base, WIN)])


@functools.lru_cache(maxsize=None)
def _target_gather_kernel():
    return functools.partial(
        pl.kernel,
        out_type=tuple(jax.ShapeDtypeStruct((T, 8), F32) for _ in range(4)),
        mesh=_sc_mesh(),
        compiler_params=pltpu.CompilerParams(use_tc_tiling_on_sc=False),
        scratch_types=(pltpu.VMEM((TCH,), I32),
                       pltpu.VMEM((TCH, 8), F32),
                       pltpu.VMEM((TCH, 8), F32),
                       pltpu.VMEM((TCH, 8), F32),
                       pltpu.VMEM((TCH, 8), F32),
                       pltpu.SemaphoreType.DMA),
    )(_target_gather_body)


def _target_gather_body(x_hbm, a1_hbm, a2_hbm, a3_hbm, tgt_hbm,
                        x_out, t1_out, t2_out, t3_out,
                        idx_v, x_v, t1_v, t2_v, t3_v, sem):
    wid = lax.axis_index("s") * 2 + lax.axis_index("c")
    base = wid * TCH
    pltpu.sync_copy(tgt_hbm.at[pl.ds(base, TCH)], idx_v)
    cps = [pltpu.async_copy(x_hbm.at[idx_v], x_v, sem),
           pltpu.async_copy(a1_hbm.at[idx_v], t1_v, sem),
           pltpu.async_copy(a2_hbm.at[idx_v], t2_v, sem),
           pltpu.async_copy(a3_hbm.at[idx_v], t3_v, sem)]
    for cp in cps:
        cp.wait()
    pltpu.sync_copy(x_v, x_out.at[pl.ds(base, TCH)])
    pltpu.sync_copy(t1_v, t1_out.at[pl.ds(base, TCH)])
    pltpu.sync_copy(t2_v, t2_out.at[pl.ds(base, TCH)])
    pltpu.sync_copy(t3_v, t3_out.at[pl.ds(base, TCH)])


def _softplus(x):
    return jnp.maximum(x, 0.0) + jnp.log(1.0 + jnp.exp(-jnp.abs(x)))


def _dist_body(z0_ref, z1_ref, z2_ref, z3_ref, bsum_ref, out_ref):
    y = (z0_ref[...] + z1_ref[...] + z2_ref[...] + z3_ref[...]
         + bsum_ref[...])[:, 0:2]
    m = jnp.max(y, axis=-1, keepdims=True)
    e = jnp.exp(y - m)
    out_ref[...] = e / jnp.sum(e, axis=-1, keepdims=True)


def kernel(X, pos, target, conv_W, conv_b, norm_g, norm_b,
           dist_W1, dist_b1, dist_W2, dist_b2):
    n, P, C = X.shape
    DIMS = pos.shape[-1]
    Xc0 = X.reshape(n, P * C)

    pos32 = pos.astype(I32)
    offset = jnp.array([1, 3, 3], I32)
    shifts = (jnp.arange(DIMS) * P).astype(I32)
    perms = []
    for i in range(DIMS):
        I = jnp.roll(pos32, shift=i, axis=-1) // offset
        key = jnp.sum(I << shifts, axis=-1)
        perms.append(jnp.argsort(key).astype(I32))

    S = n
    ar = jnp.arange(NTOT, dtype=I32)
    head = jnp.full((8,), S, I32)
    tail = jnp.full((PEXT - 8 - n,), S, I32)
    pexts = [jnp.concatenate([head, p, tail]) for p in perms]
    l1, r1, l2, r2, l3, r3 = _nbr_kernel()(*pexts)
    idx_list = [jnp.where((ar >= 1) & (ar < n), ar - 1, S),
                jnp.where(ar < n - 1, ar + 1, S),
                l1, r1, l2, r2, l3, r3]
    idxp = jnp.stack(idx_list).reshape(8, NTOT // WIN, WIN)
    idxp = idxp.transpose(1, 0, 2).reshape(-1)

    w2p = [jnp.pad(w, ((0, 0), (0, 6))) for w in dist_W2]
    zs = []
    x = Xc0
    for i in range(3):
        wcat = jnp.concatenate([conv_W[i][0], conv_W[i][1], conv_W[i][2]],
                               axis=-1)
        cb = conv_b[i].reshape(1, KK)
        if i == 0:
            wcat = jnp.concatenate([wcat, dist_W1[0]], axis=-1)
            a, bb, c, z = _proj(x, wcat, cb, db=dist_b1[0].reshape(1, 64),
                                w2h=w2p[0])
        else:
            a, bb, c, z = _proj(x, wcat, cb, norm_g[i - 1].reshape(1, KK),
                                norm_b[i - 1].reshape(1, KK),
                                w1h=dist_W1[i], b1h=dist_b1[i].reshape(1, 64),
                                w2h=w2p[i])
        zs.append(z)
        acc = _gather_acc_kernel()(a, c, bb, idxp)
        x = acc
    zs.append(_final_head(x, norm_g[2].reshape(1, KK),
                          norm_b[2].reshape(1, KK), dist_W1[3],
                          dist_b1[3].reshape(1, 64), w2p[3]))

    z0g, z1g, z2g, z3g = _target_gather_kernel()(zs[0], zs[1], zs[2], zs[3],
                                                 target)
    bsum = (dist_b2[0] + dist_b2[1] + dist_b2[2] + dist_b2[3])
    bsum = jnp.pad(bsum, (0, 6)).reshape(1, 8)
    grid = T // BT
    blk = lambda c: pl.BlockSpec((BT, c), lambda m: (m, 0))
    out = pl.pallas_call(
        _dist_body, grid=(grid,),
        in_specs=[blk(8)] * 4 + [pl.BlockSpec((1, 8), lambda m: (0, 0))],
        out_specs=blk(2),
        out_shape=jax.ShapeDtypeStruct((T, 2), F32),
    )(z0g, z1g, z2g, z3g, bsum)
    return out
